# baseline (speedup 1.0000x reference)
import functools

import jax
import jax.numpy as jnp
from jax.experimental import pallas as pl
from jax.experimental.pallas import tpu as pltpu

_G = 16
_K = 3
_BR = 320
_BC = 1024
_BIG = 2 ** 30


def _edge_conv_kernel(F, Fo, Bc, cs_ref, ce_ref, x_ref, xt_ref, bi_ref, bc_ref,
                      W0_ref, b0_ref, W1_ref, b1_ref, W2_ref, b2_ref, out_ref):
    r = pl.program_id(0)
    Br = out_ref.shape[0]
    inf = jnp.float32(jnp.inf)

    xi = x_ref[pl.ds(r * Br, Br), :]
    bi = bi_ref[pl.ds(r * Br, Br), :]

    d_init = jnp.full((Br, 1), inf, jnp.float32)
    f_init = jnp.zeros((Br, F), jnp.float32)
    carry0 = (d_init, f_init, d_init, f_init, d_init, f_init)

    cs = cs_ref[r]

    def body(c, carry):
        d0, f0, d1, f1, d2, f2 = carry
        s = pl.multiple_of(cs + c * Bc, 128)
        xc = x_ref[pl.ds(s, Bc), :]
        xct = xt_ref[:, pl.ds(s, Bc)]
        bcv = bc_ref[:, pl.ds(s, Bc)]
        sq_c = jnp.sum(xct * xct, axis=0, keepdims=True)
        dots = jnp.dot(xi, xct, preferred_element_type=jnp.float32)
        d = sq_c - 2.0 * dots
        d = jnp.where(bi != bcv, inf, d)
        for _ in range(_K):
            m = jnp.min(d, axis=1, keepdims=True)
            eq = d == m
            fj = jnp.dot(eq.astype(jnp.float32), xc,
                         preferred_element_type=jnp.float32)
            d = jnp.where(eq, inf, d)
            w0 = m < d0
            w1 = m < d1
            w2 = m < d2
            d2 = jnp.where(w1, d1, jnp.where(w2, m, d2))
            f2 = jnp.where(w1, f1, jnp.where(w2, fj, f2))
            d1 = jnp.where(w0, d0, jnp.where(w1, m, d1))
            f1 = jnp.where(w0, f0, jnp.where(w1, fj, f1))
            d0 = jnp.where(w0, m, d0)
            f0 = jnp.where(w0, fj, f0)
        return (d0, f0, d1, f1, d2, f2)

    carry = body(0, carry0)
    carry = jax.lax.cond(ce_ref[r] > 1, lambda cc: body(1, cc),
                         lambda cc: cc, carry)
    carry = jax.lax.fori_loop(2, ce_ref[r], body, carry)
    _, f0, _, f1, _, f2 = carry

    W0a = W0_ref[:F, :]
    W0b = W0_ref[F:, :]
    base = jnp.dot(xi, W0a, preferred_element_type=jnp.float32) + b0_ref[:, :]
    cat = jnp.concatenate([f0 - xi, f1 - xi, f2 - xi], axis=0)
    base3 = jnp.concatenate([base, base, base], axis=0)
    h = jax.nn.relu(base3 + jnp.dot(cat, W0b,
                                    preferred_element_type=jnp.float32))
    h = jax.nn.relu(jnp.dot(h, W1_ref[:, :],
                            preferred_element_type=jnp.float32) + b1_ref[:, :])
    h = jax.nn.relu(jnp.dot(h, W2_ref[:, :],
                            preferred_element_type=jnp.float32) + b2_ref[:, :])
    acc = h[:Br, :] + h[Br:2 * Br, :] + h[2 * Br:, :]
    out_ref[:, :] = acc * jnp.float32(1.0 / 3.0)


def _edge_conv(xp, bi, bc, cs, ce, W0, b0, W1, b1, W2, b2, Br, Bc):
    Npad, F = xp.shape
    Fo = W2.shape[1]
    R = Npad // Br
    full = lambda shape: pl.BlockSpec(shape, lambda r, *_: (0,) * len(shape))
    grid_spec = pltpu.PrefetchScalarGridSpec(
        num_scalar_prefetch=2,
        grid=(R,),
        in_specs=[
            full((Npad, F)),
            full((F, Npad)),
            full((Npad, 1)),
            full((1, Npad)),
            full(W0.shape), full(b0.shape),
            full(W1.shape), full(b1.shape),
            full(W2.shape), full(b2.shape),
        ],
        out_specs=pl.BlockSpec((Br, Fo), lambda r, *_: (r, 0)),
    )
    return pl.pallas_call(
        functools.partial(_edge_conv_kernel, F, Fo, Bc),
        grid_spec=grid_spec,
        out_shape=jax.ShapeDtypeStruct((Npad, Fo), jnp.float32),
        compiler_params=pltpu.CompilerParams(
            dimension_semantics=("parallel",)),
    )(cs, ce, xp, xp.T, bi, bc, W0, b0, W1, b1, W2, b2)


def _pool_kernel(G, h_ref, bc_ref, W0_ref, b0_ref, W1_ref, b1_ref, out_ref):
    h = h_ref[:, :]
    bcv = bc_ref[:, :]
    gi = jax.lax.broadcasted_iota(jnp.int32, (G, h.shape[0]), 0)
    oh = (gi == bcv).astype(jnp.float32)
    sums = jnp.dot(oh, h, preferred_element_type=jnp.float32)
    cnt = jnp.sum(oh, axis=1, keepdims=True)
    pooled = sums / jnp.maximum(cnt, 1.0)
    z = jax.nn.relu(jnp.dot(pooled, W0_ref[:, :],
                            preferred_element_type=jnp.float32) + b0_ref[:, :])
    out_ref[:, :] = jnp.dot(z, W1_ref[:, :],
                            preferred_element_type=jnp.float32) + b1_ref[:, :]


def _pool_head(h3, bc, W0, b0, W1, b1, G):
    return pl.pallas_call(
        functools.partial(_pool_kernel, G),
        out_shape=jax.ShapeDtypeStruct((G, W1.shape[1]), jnp.float32),
    )(h3, bc, W0, b0, W1, b1)


def _gnn(x, batch, params, head, G, Br, Bc):
    N = x.shape[0]
    Npad = -(-N // Br) * Br
    xp = jnp.zeros((Npad, x.shape[1]), jnp.float32).at[:N].set(x)
    bpad = jnp.full((Npad,), G, jnp.int32).at[:N].set(batch.astype(jnp.int32))
    bi = bpad.reshape(Npad, 1)
    bc = bpad.reshape(1, Npad)

    R = Npad // Br
    starts = bpad[jnp.arange(R) * Br]
    ends = bpad[jnp.arange(1, R + 1) * Br - 1]
    lo = (jnp.searchsorted(bpad, starts, side="left") // 128) * 128
    hi = jnp.searchsorted(bpad, ends, side="right")
    nb = -(-(hi - lo) // Bc)
    nb = jnp.where(starts == G, 0, nb)
    lo = jnp.minimum(lo, Npad - jnp.maximum(nb, 1) * Bc)
    cs = lo.astype(jnp.int32)
    ce = nb.astype(jnp.int32)

    h = xp
    for (W0, b0, W1, b1, W2, b2) in params:
        h = _edge_conv(h, bi, bc, cs, ce,
                       W0, b0.reshape(1, -1), W1, b1.reshape(1, -1),
                       W2, b2.reshape(1, -1), Br, Bc)
    mf_W0, mf_b0, mf_W1, mf_b1 = head
    return _pool_head(h, bc, mf_W0, mf_b0.reshape(1, -1),
                      mf_W1, mf_b1.reshape(1, -1), G)


def kernel(x, batch, m1_W0, m1_b0, m1_W1, m1_b1, m1_W2, m1_b2,
           m2_W0, m2_b0, m2_W1, m2_b1, m2_W2, m2_b2,
           m3_W0, m3_b0, m3_W1, m3_b1, m3_W2, m3_b2,
           mf_W0, mf_b0, mf_W1, mf_b1):
    params = [
        (m1_W0, m1_b0, m1_W1, m1_b1, m1_W2, m1_b2),
        (m2_W0, m2_b0, m2_W1, m2_b1, m2_W2, m2_b2),
        (m3_W0, m3_b0, m3_W1, m3_b1, m3_W2, m3_b2),
    ]
    head = (mf_W0, mf_b0, mf_W1, mf_b1)
    return _gnn(x, batch, params, head, _G, _BR, _BC)


# speedup vs baseline: 1.0396x; 1.0396x over previous
import functools

import jax
import jax.numpy as jnp
from jax.experimental import pallas as pl
from jax.experimental.pallas import tpu as pltpu

_G = 16
_K = 3
_BR = 320
_BC = 1024
_BIG = 2 ** 30


def _edge_conv_kernel(F, Fo, Bc, cs_ref, ce_ref, x_ref, xt_ref, bi_ref, bc_ref,
                      W0_ref, b0_ref, W1_ref, b1_ref, W2_ref, b2_ref, out_ref):
    r = pl.program_id(0)
    Br = out_ref.shape[0]
    inf = jnp.float32(jnp.inf)

    xi = x_ref[pl.ds(r * Br, Br), :]
    bi = bi_ref[pl.ds(r * Br, Br), :]

    d_init = jnp.full((Br, 1), inf, jnp.float32)
    f_init = jnp.zeros((Br, F), jnp.float32)
    carry0 = (d_init, f_init, d_init, f_init, d_init, f_init)

    cs = cs_ref[r]

    def body(c, carry):
        d0, f0, d1, f1, d2, f2 = carry
        s = pl.multiple_of(cs + c * Bc, 128)
        xc = x_ref[pl.ds(s, Bc), :]
        xct = xt_ref[:, pl.ds(s, Bc)]
        bcv = bc_ref[:, pl.ds(s, Bc)]
        sq_c = jnp.sum(xct * xct, axis=0, keepdims=True)
        dots = jnp.dot(xi, xct, preferred_element_type=jnp.float32)
        d = sq_c - 2.0 * dots
        d = jnp.where(bi != bcv, inf, d)
        for _ in range(_K):
            m = jnp.min(d, axis=1, keepdims=True)
            eq = d == m
            fj = jnp.dot(eq.astype(jnp.float32), xc,
                         preferred_element_type=jnp.float32)
            d = jnp.where(eq, inf, d)
            w0 = m < d0
            w1 = m < d1
            w2 = m < d2
            d2 = jnp.where(w1, d1, jnp.where(w2, m, d2))
            f2 = jnp.where(w1, f1, jnp.where(w2, fj, f2))
            d1 = jnp.where(w0, d0, jnp.where(w1, m, d1))
            f1 = jnp.where(w0, f0, jnp.where(w1, fj, f1))
            d0 = jnp.where(w0, m, d0)
            f0 = jnp.where(w0, fj, f0)
        return (d0, f0, d1, f1, d2, f2)

    carry = body(0, carry0)
    carry = jax.lax.fori_loop(1, ce_ref[r], body, carry)
    _, f0, _, f1, _, f2 = carry

    W0a = W0_ref[:F, :]
    W0b = W0_ref[F:, :]
    base = jnp.dot(xi, W0a, preferred_element_type=jnp.float32) + b0_ref[:, :]
    cat = jnp.concatenate([f0 - xi, f1 - xi, f2 - xi], axis=0)
    base3 = jnp.concatenate([base, base, base], axis=0)
    h = jax.nn.relu(base3 + jnp.dot(cat, W0b,
                                    preferred_element_type=jnp.float32))
    h = jax.nn.relu(jnp.dot(h, W1_ref[:, :],
                            preferred_element_type=jnp.float32) + b1_ref[:, :])
    h = jax.nn.relu(jnp.dot(h, W2_ref[:, :],
                            preferred_element_type=jnp.float32) + b2_ref[:, :])
    acc = h[:Br, :] + h[Br:2 * Br, :] + h[2 * Br:, :]
    out_ref[:, :] = acc * jnp.float32(1.0 / 3.0)


def _edge_conv(xp, bi, bc, cs, ce, W0, b0, W1, b1, W2, b2, Br, Bc):
    Npad, F = xp.shape
    Fo = W2.shape[1]
    R = Npad // Br
    full = lambda shape: pl.BlockSpec(shape, lambda r, *_: (0,) * len(shape))
    grid_spec = pltpu.PrefetchScalarGridSpec(
        num_scalar_prefetch=2,
        grid=(R,),
        in_specs=[
            full((Npad, F)),
            full((F, Npad)),
            full((Npad, 1)),
            full((1, Npad)),
            full(W0.shape), full(b0.shape),
            full(W1.shape), full(b1.shape),
            full(W2.shape), full(b2.shape),
        ],
        out_specs=pl.BlockSpec((Br, Fo), lambda r, *_: (r, 0)),
    )
    return pl.pallas_call(
        functools.partial(_edge_conv_kernel, F, Fo, Bc),
        grid_spec=grid_spec,
        out_shape=jax.ShapeDtypeStruct((Npad, Fo), jnp.float32),
        compiler_params=pltpu.CompilerParams(
            dimension_semantics=("parallel",)),
    )(cs, ce, xp, xp.T, bi, bc, W0, b0, W1, b1, W2, b2)


def _pool_kernel(G, h_ref, bc_ref, W0_ref, b0_ref, W1_ref, b1_ref, out_ref):
    h = h_ref[:, :]
    bcv = bc_ref[:, :]
    gi = jax.lax.broadcasted_iota(jnp.int32, (G, h.shape[0]), 0)
    oh = (gi == bcv).astype(jnp.float32)
    sums = jnp.dot(oh, h, preferred_element_type=jnp.float32)
    cnt = jnp.sum(oh, axis=1, keepdims=True)
    pooled = sums / jnp.maximum(cnt, 1.0)
    z = jax.nn.relu(jnp.dot(pooled, W0_ref[:, :],
                            preferred_element_type=jnp.float32) + b0_ref[:, :])
    out_ref[:, :] = jnp.dot(z, W1_ref[:, :],
                            preferred_element_type=jnp.float32) + b1_ref[:, :]


def _pool_head(h3, bc, W0, b0, W1, b1, G):
    return pl.pallas_call(
        functools.partial(_pool_kernel, G),
        out_shape=jax.ShapeDtypeStruct((G, W1.shape[1]), jnp.float32),
    )(h3, bc, W0, b0, W1, b1)


def _gnn(x, batch, params, head, G, Br, Bc):
    N = x.shape[0]
    Npad = -(-N // Br) * Br
    xp = jnp.zeros((Npad, x.shape[1]), jnp.float32).at[:N].set(x)
    bpad = jnp.full((Npad,), G, jnp.int32).at[:N].set(batch.astype(jnp.int32))
    bi = bpad.reshape(Npad, 1)
    bc = bpad.reshape(1, Npad)

    R = Npad // Br
    starts = bpad[jnp.arange(R) * Br]
    ends = bpad[jnp.arange(1, R + 1) * Br - 1]
    lo = (jnp.searchsorted(bpad, starts, side="left") // 128) * 128
    hi = jnp.searchsorted(bpad, ends, side="right")
    nb = -(-(hi - lo) // Bc)
    nb = jnp.where(starts == G, 0, nb)
    lo = jnp.minimum(lo, Npad - jnp.maximum(nb, 1) * Bc)
    cs = lo.astype(jnp.int32)
    ce = nb.astype(jnp.int32)

    h = xp
    for (W0, b0, W1, b1, W2, b2) in params:
        h = _edge_conv(h, bi, bc, cs, ce,
                       W0, b0.reshape(1, -1), W1, b1.reshape(1, -1),
                       W2, b2.reshape(1, -1), Br, Bc)
    mf_W0, mf_b0, mf_W1, mf_b1 = head
    return _pool_head(h, bc, mf_W0, mf_b0.reshape(1, -1),
                      mf_W1, mf_b1.reshape(1, -1), G)


def kernel(x, batch, m1_W0, m1_b0, m1_W1, m1_b1, m1_W2, m1_b2,
           m2_W0, m2_b0, m2_W1, m2_b1, m2_W2, m2_b2,
           m3_W0, m3_b0, m3_W1, m3_b1, m3_W2, m3_b2,
           mf_W0, mf_b0, mf_W1, mf_b1):
    params = [
        (m1_W0, m1_b0, m1_W1, m1_b1, m1_W2, m1_b2),
        (m2_W0, m2_b0, m2_W1, m2_b1, m2_W2, m2_b2),
        (m3_W0, m3_b0, m3_W1, m3_b1, m3_W2, m3_b2),
    ]
    head = (mf_W0, mf_b0, mf_W1, mf_b1)
    return _gnn(x, batch, params, head, _G, _BR, _BC)


# speedup vs baseline: 1.0402x; 1.0006x over previous
import functools

import jax
import jax.numpy as jnp
from jax.experimental import pallas as pl
from jax.experimental.pallas import tpu as pltpu

_G = 16
_K = 3
_BR = 320
_BC = 1024


def _edge_conv_kernel(F, Fo, Bc, cs_ref, ce_ref, x_ref, xt_ref, bi_ref, bc_ref,
                      W0_ref, b0_ref, W1_ref, b1_ref, W2_ref, b2_ref, out_ref):
    r = pl.program_id(0)
    Br = out_ref.shape[0]
    inf = jnp.float32(jnp.inf)

    xi = x_ref[pl.ds(r * Br, Br), :]
    bi = bi_ref[pl.ds(r * Br, Br), :]

    d_init = jnp.full((Br, 1), inf, jnp.float32)
    f_init = jnp.zeros((Br, F), jnp.float32)
    carry0 = (d_init, f_init, d_init, f_init, d_init, f_init)

    cs = cs_ref[r]

    def body(c, carry):
        d0, f0, d1, f1, d2, f2 = carry
        s = pl.multiple_of(cs + c * Bc, 128)
        xc = x_ref[pl.ds(s, Bc), :]
        xct = xt_ref[:, pl.ds(s, Bc)]
        bcv = bc_ref[:, pl.ds(s, Bc)]
        sq_c = jnp.sum(xct * xct, axis=0, keepdims=True)
        dots = jnp.dot(xi, xct, preferred_element_type=jnp.float32)
        d = sq_c - 2.0 * dots
        d = jnp.where(bi != bcv, inf, d)
        for _ in range(_K):
            m = jnp.min(d, axis=1, keepdims=True)
            eq = d == m
            fj = jnp.dot(eq.astype(jnp.float32), xc,
                         preferred_element_type=jnp.float32)
            d = jnp.where(eq, inf, d)
            w0 = m < d0
            w1 = m < d1
            w2 = m < d2
            d2 = jnp.where(w1, d1, jnp.where(w2, m, d2))
            f2 = jnp.where(w1, f1, jnp.where(w2, fj, f2))
            d1 = jnp.where(w0, d0, jnp.where(w1, m, d1))
            f1 = jnp.where(w0, f0, jnp.where(w1, fj, f1))
            d0 = jnp.where(w0, m, d0)
            f0 = jnp.where(w0, fj, f0)
        return (d0, f0, d1, f1, d2, f2)

    carry = body(0, carry0)
    carry = jax.lax.fori_loop(1, ce_ref[r], body, carry)
    _, f0, _, f1, _, f2 = carry

    W0a = W0_ref[:F, :]
    W0b = W0_ref[F:, :]
    base = jnp.dot(xi, W0a, preferred_element_type=jnp.float32) + b0_ref[:, :]
    cat = jnp.concatenate([f0 - xi, f1 - xi, f2 - xi], axis=0)
    base3 = jnp.concatenate([base, base, base], axis=0)
    h = jax.nn.relu(base3 + jnp.dot(cat, W0b,
                                    preferred_element_type=jnp.float32))
    h = jax.nn.relu(jnp.dot(h, W1_ref[:, :],
                            preferred_element_type=jnp.float32) + b1_ref[:, :])
    h = jax.nn.relu(jnp.dot(h, W2_ref[:, :],
                            preferred_element_type=jnp.float32) + b2_ref[:, :])
    acc = h[:Br, :] + h[Br:2 * Br, :] + h[2 * Br:, :]
    out_ref[:, :] = acc * jnp.float32(1.0 / 3.0)


def _edge_conv(xp, bi, bc, cs, ce, W0, b0, W1, b1, W2, b2, Br, Bc):
    Npad, F = xp.shape
    Fo = W2.shape[1]
    R = Npad // Br
    full = lambda shape: pl.BlockSpec(shape, lambda r, *_: (0,) * len(shape))
    grid_spec = pltpu.PrefetchScalarGridSpec(
        num_scalar_prefetch=2,
        grid=(R,),
        in_specs=[
            full((Npad, F)),
            full((F, Npad)),
            full((Npad, 1)),
            full((1, Npad)),
            full(W0.shape), full(b0.shape),
            full(W1.shape), full(b1.shape),
            full(W2.shape), full(b2.shape),
        ],
        out_specs=pl.BlockSpec((Br, Fo), lambda r, *_: (r, 0)),
    )
    return pl.pallas_call(
        functools.partial(_edge_conv_kernel, F, Fo, Bc),
        grid_spec=grid_spec,
        out_shape=jax.ShapeDtypeStruct((Npad, Fo), jnp.float32),
        compiler_params=pltpu.CompilerParams(
            dimension_semantics=("parallel",)),
    )(cs, ce, xp, xp.T, bi, bc, W0, b0, W1, b1, W2, b2)


def _pool_kernel(G, h_ref, bc_ref, W0_ref, b0_ref, W1_ref, b1_ref, out_ref):
    h = h_ref[:, :]
    bcv = bc_ref[:, :]
    gi = jax.lax.broadcasted_iota(jnp.int32, (G, h.shape[0]), 0)
    oh = (gi == bcv).astype(jnp.float32)
    sums = jnp.dot(oh, h, preferred_element_type=jnp.float32)
    cnt = jnp.sum(oh, axis=1, keepdims=True)
    pooled = sums / jnp.maximum(cnt, 1.0)
    z = jax.nn.relu(jnp.dot(pooled, W0_ref[:, :],
                            preferred_element_type=jnp.float32) + b0_ref[:, :])
    out_ref[:, :] = jnp.dot(z, W1_ref[:, :],
                            preferred_element_type=jnp.float32) + b1_ref[:, :]


def _pool_head(h3, bc, W0, b0, W1, b1, G):
    return pl.pallas_call(
        functools.partial(_pool_kernel, G),
        out_shape=jax.ShapeDtypeStruct((G, W1.shape[1]), jnp.float32),
    )(h3, bc, W0, b0, W1, b1)


def _gnn(x, batch, params, head, G, Br, Bc):
    N = x.shape[0]
    Npad = -(-N // Br) * Br
    xp = jnp.zeros((Npad, x.shape[1]), jnp.float32).at[:N].set(x)
    bpad = jnp.full((Npad,), G, jnp.int32).at[:N].set(batch.astype(jnp.int32))
    bi = bpad.reshape(Npad, 1)
    bc = bpad.reshape(1, Npad)

    R = Npad // Br
    starts = bpad[jnp.arange(R) * Br]
    ends = bpad[jnp.arange(1, R + 1) * Br - 1]
    lo = (jnp.searchsorted(bpad, starts, side="left") // 128) * 128
    hi = jnp.searchsorted(bpad, ends, side="right")
    nb = -(-(hi - lo) // Bc)
    nb = jnp.where(starts == G, 0, nb)
    lo = jnp.minimum(lo, Npad - jnp.maximum(nb, 1) * Bc)
    cs = lo.astype(jnp.int32)
    ce = nb.astype(jnp.int32)

    h = xp
    for (W0, b0, W1, b1, W2, b2) in params:
        h = _edge_conv(h, bi, bc, cs, ce,
                       W0, b0.reshape(1, -1), W1, b1.reshape(1, -1),
                       W2, b2.reshape(1, -1), Br, Bc)
    mf_W0, mf_b0, mf_W1, mf_b1 = head
    return _pool_head(h, bc, mf_W0, mf_b0.reshape(1, -1),
                      mf_W1, mf_b1.reshape(1, -1), G)


def kernel(x, batch, m1_W0, m1_b0, m1_W1, m1_b1, m1_W2, m1_b2,
           m2_W0, m2_b0, m2_W1, m2_b1, m2_W2, m2_b2,
           m3_W0, m3_b0, m3_W1, m3_b1, m3_W2, m3_b2,
           mf_W0, mf_b0, mf_W1, mf_b1):
    params = [
        (m1_W0, m1_b0, m1_W1, m1_b1, m1_W2, m1_b2),
        (m2_W0, m2_b0, m2_W1, m2_b1, m2_W2, m2_b2),
        (m3_W0, m3_b0, m3_W1, m3_b1, m3_W2, m3_b2),
    ]
    head = (mf_W0, mf_b0, mf_W1, mf_b1)
    return _gnn(x, batch, params, head, _G, _BR, _BC)
